# baseline (speedup 1.0000x reference)
import functools

import jax
import jax.numpy as jnp
from jax import lax
from jax.experimental import pallas as pl
from jax.experimental.pallas import tpu as pltpu
from jax.experimental.pallas import tpu_sc as plsc

N = 65536
C = 80
TOTAL = N * C
L = 16
VPR = C // L

R_SC = 8192
R_TC = N - R_SC
NW = 32
RPW = R_SC // NW
CROWS = 128
NCH = RPW // CROWS

BR = 4096
G_TC = R_TC // BR
SUB = 32

_LOG1P_C = (1.6936626598407223e-06, 0.9998325947816316, -0.49720333122019134,
            0.31504127990864345, -0.18901954822291905, 0.08152317761736225,
            -0.017029610589052675)


def _log1p01(u):
    p = jnp.float32(_LOG1P_C[6])
    for c in _LOG1P_C[5::-1]:
        p = p * u + jnp.float32(c)
    return p


def _pow15(a):
    i = lax.bitcast_convert_type(a, jnp.int32)
    y = lax.bitcast_convert_type(
        jnp.int32(0x5F3759DF) - lax.shift_right_arithmetic(i, 1), jnp.float32)
    y = y * (1.5 - 0.5 * a * y * y)
    y = y * (1.5 - 0.5 * a * y * y)
    return a * a * y


def _elem(x, pos, sc):
    ax = jnp.abs(x)
    u = jnp.exp(-ax)
    d = 1.0 / (1.0 + u)
    sp = jnp.maximum(x, 0.0) + _log1p01(u)
    s = jnp.where(x >= 0.0, d, 1.0 - d)
    scm = jnp.where(pos, sc, 0.0)
    a = jnp.where(pos, jnp.abs(sc - s), s)
    return (sp - x * scm) * _pow15(a)


def _sc_body(pred_h, lab_h, score_h, out_h,
             pb0, pb1, lb0, lb1, sc_v, acc_v,
             sp0, sp1, sl0, sl1):
    wid = lax.axis_index("s") * 2 + lax.axis_index("c")
    base = wid * RPW
    last = base + (NCH - 1) * CROWS

    pltpu.sync_copy(score_h, sc_v)
    scv = [sc_v[pl.ds(L * v, L)] for v in range(VPR)]

    def start(row0, pb, lb, sp, sl):
        row = jnp.minimum(row0, last)
        pltpu.async_copy(pred_h.at[pl.ds(row, CROWS)], pb, sp)
        pltpu.async_copy(lab_h.at[pl.ds(row, CROWS)], lb, sl)

    def wait(pb, lb, sp, sl):
        pltpu.make_async_copy(pred_h.at[pl.ds(base, CROWS)], pb, sp).wait()
        pltpu.make_async_copy(lab_h.at[pl.ds(base, CROWS)], lb, sl).wait()

    def compute(pb, lb, acc):
        def row(r, acc):
            for v in range(VPR):
                x = pb[r, pl.ds(L * v, L)]
                lv = lb[r, pl.ds(L * v, L)]
                acc = acc + _elem(x, lv > 0, scv[v])
            return acc
        return lax.fori_loop(0, CROWS, row, acc)

    start(base, pb0, lb0, sp0, sl0)
    start(base + CROWS, pb1, lb1, sp1, sl1)

    def pair(g, acc):
        c0 = base + (2 * g) * CROWS
        wait(pb0, lb0, sp0, sl0)
        acc = compute(pb0, lb0, acc)
        start(c0 + 2 * CROWS, pb0, lb0, sp0, sl0)
        wait(pb1, lb1, sp1, sl1)
        acc = compute(pb1, lb1, acc)
        start(c0 + 3 * CROWS, pb1, lb1, sp1, sl1)
        return acc

    acc = lax.fori_loop(0, NCH // 2, pair, jnp.zeros((L,), jnp.float32))

    if NCH % 2:
        wait(pb0, lb0, sp0, sl0)
        acc = compute(pb0, lb0, acc)
        start(last, pb0, lb0, sp0, sl0)

    wait(pb0, lb0, sp0, sl0)
    wait(pb1, lb1, sp1, sl1)

    acc_v[...] = acc
    pltpu.sync_copy(acc_v, out_h.at[pl.ds(wid * L, L)])


def _sc_call(pred_sc, lab_sc, score):
    mesh = plsc.VectorSubcoreMesh(core_axis_name="c", subcore_axis_name="s")
    f = functools.partial(
        pl.kernel,
        mesh=mesh,
        out_type=jax.ShapeDtypeStruct((NW * L,), jnp.float32),
        scratch_types=[
            pltpu.VMEM((CROWS, C), jnp.float32),
            pltpu.VMEM((CROWS, C), jnp.float32),
            pltpu.VMEM((CROWS, C), jnp.int32),
            pltpu.VMEM((CROWS, C), jnp.int32),
            pltpu.VMEM((C,), jnp.float32),
            pltpu.VMEM((L,), jnp.float32),
            pltpu.SemaphoreType.DMA,
            pltpu.SemaphoreType.DMA,
            pltpu.SemaphoreType.DMA,
            pltpu.SemaphoreType.DMA,
        ],
    )(_sc_body)
    return f(pred_sc, lab_sc, score)


def _tc_body(score_ref, pred_ref, lab_ref, out_ref):
    sc = score_ref[...]

    def sub(j, acc):
        r0 = pl.multiple_of(j * SUB, SUB)
        x = pred_ref[pl.ds(r0, SUB), :]
        pos = lab_ref[pl.ds(r0, SUB), :] > 0
        ax = jnp.abs(x)
        u = jnp.exp(-ax)
        d = 1.0 / (1.0 + u)
        sp = jnp.maximum(x, 0.0) + jnp.log1p(u)
        s = jnp.where(x >= 0.0, d, 1.0 - d)
        scm = jnp.where(pos, sc, 0.0)
        a = jnp.where(pos, jnp.abs(sc - s), s)
        return acc + (sp - x * scm) * (a * lax.sqrt(a))

    acc = lax.fori_loop(0, BR // SUB, sub, jnp.zeros((SUB, C), jnp.float32))

    @pl.when(pl.program_id(0) == 0)
    def _():
        out_ref[0, 0] = 0.0

    out_ref[0, 0] += jnp.sum(acc)


def _tc_call(pred, label, score):
    return pl.pallas_call(
        _tc_body,
        grid=(G_TC,),
        in_specs=[
            pl.BlockSpec((1, C), lambda i: (0, 0)),
            pl.BlockSpec((BR, C), lambda i: (i, 0)),
            pl.BlockSpec((BR, C), lambda i: (i, 0)),
        ],
        out_specs=pl.BlockSpec((1, 1), lambda i: (0, 0),
                               memory_space=pltpu.SMEM),
        out_shape=jax.ShapeDtypeStruct((1, 1), jnp.float32),
    )(score.reshape(1, C), pred, label)


@jax.jit
def kernel(pred, label, score):
    sc_part = _sc_call(pred[R_TC:], label[R_TC:], score)
    tc_part = _tc_call(pred, label, score)
    return (jnp.sum(tc_part) + jnp.sum(sc_part)) / jnp.float32(TOTAL)


# speedup vs baseline: 1.1802x; 1.1802x over previous
import functools

import jax
import jax.numpy as jnp
from jax import lax
from jax.experimental import pallas as pl
from jax.experimental.pallas import tpu as pltpu
from jax.experimental.pallas import tpu_sc as plsc

N = 65536
C = 80
TOTAL = N * C
L = 16
VPR = C // L

R_SC = 8192
R_TC = N - R_SC
NW = 32
RPW = R_SC // NW
CROWS = 128
NCH = RPW // CROWS

BR = 2048
G_TC = R_TC // BR
SUB = 32

_LOG1P_C = (1.6936626598407223e-06, 0.9998325947816316, -0.49720333122019134,
            0.31504127990864345, -0.18901954822291905, 0.08152317761736225,
            -0.017029610589052675)


def _log1p01(u):
    p = jnp.float32(_LOG1P_C[6])
    for c in _LOG1P_C[5::-1]:
        p = p * u + jnp.float32(c)
    return p


def _pow15(a):
    i = lax.bitcast_convert_type(a, jnp.int32)
    y = lax.bitcast_convert_type(
        jnp.int32(0x5F3759DF) - lax.shift_right_arithmetic(i, 1), jnp.float32)
    y = y * (1.5 - 0.5 * a * y * y)
    y = y * (1.5 - 0.5 * a * y * y)
    return a * a * y


def _elem(x, pos, sc):
    ax = jnp.abs(x)
    u = jnp.exp(-ax)
    d = 1.0 / (1.0 + u)
    sp = jnp.maximum(x, 0.0) + _log1p01(u)
    s = jnp.where(x >= 0.0, d, 1.0 - d)
    scm = jnp.where(pos, sc, 0.0)
    a = jnp.where(pos, jnp.abs(sc - s), s)
    return (sp - x * scm) * _pow15(a)


def _sc_body(pred_h, lab_h, score_h, out_h,
             pb0, pb1, lb0, lb1, sc_v, acc_v,
             sp0, sp1, sl0, sl1):
    wid = lax.axis_index("s") * 2 + lax.axis_index("c")
    base = wid * RPW
    last = base + (NCH - 1) * CROWS

    pltpu.sync_copy(score_h, sc_v)
    scv = [sc_v[pl.ds(L * v, L)] for v in range(VPR)]

    def start(row0, pb, lb, sp, sl):
        row = jnp.minimum(row0, last)
        pltpu.async_copy(pred_h.at[pl.ds(row, CROWS)], pb, sp)
        pltpu.async_copy(lab_h.at[pl.ds(row, CROWS)], lb, sl)

    def wait(pb, lb, sp, sl):
        pltpu.make_async_copy(pred_h.at[pl.ds(base, CROWS)], pb, sp).wait()
        pltpu.make_async_copy(lab_h.at[pl.ds(base, CROWS)], lb, sl).wait()

    def compute(pb, lb, acc):
        def row(r, acc):
            for v in range(VPR):
                x = pb[r, pl.ds(L * v, L)]
                lv = lb[r, pl.ds(L * v, L)]
                acc = acc + _elem(x, lv > 0, scv[v])
            return acc
        return lax.fori_loop(0, CROWS, row, acc)

    start(base, pb0, lb0, sp0, sl0)
    start(base + CROWS, pb1, lb1, sp1, sl1)

    def pair(g, acc):
        c0 = base + (2 * g) * CROWS
        wait(pb0, lb0, sp0, sl0)
        acc = compute(pb0, lb0, acc)
        start(c0 + 2 * CROWS, pb0, lb0, sp0, sl0)
        wait(pb1, lb1, sp1, sl1)
        acc = compute(pb1, lb1, acc)
        start(c0 + 3 * CROWS, pb1, lb1, sp1, sl1)
        return acc

    acc = lax.fori_loop(0, NCH // 2, pair, jnp.zeros((L,), jnp.float32))

    if NCH % 2:
        wait(pb0, lb0, sp0, sl0)
        acc = compute(pb0, lb0, acc)
        start(last, pb0, lb0, sp0, sl0)

    wait(pb0, lb0, sp0, sl0)
    wait(pb1, lb1, sp1, sl1)

    acc_v[...] = acc
    pltpu.sync_copy(acc_v, out_h.at[pl.ds(wid * L, L)])


def _sc_call(pred_sc, lab_sc, score):
    mesh = plsc.VectorSubcoreMesh(core_axis_name="c", subcore_axis_name="s")
    f = functools.partial(
        pl.kernel,
        mesh=mesh,
        out_type=jax.ShapeDtypeStruct((NW * L,), jnp.float32),
        compiler_params=pltpu.CompilerParams(use_tc_tiling_on_sc=True),
        scratch_types=[
            pltpu.VMEM((CROWS, C), jnp.float32),
            pltpu.VMEM((CROWS, C), jnp.float32),
            pltpu.VMEM((CROWS, C), jnp.int32),
            pltpu.VMEM((CROWS, C), jnp.int32),
            pltpu.VMEM((C,), jnp.float32),
            pltpu.VMEM((L,), jnp.float32),
            pltpu.SemaphoreType.DMA,
            pltpu.SemaphoreType.DMA,
            pltpu.SemaphoreType.DMA,
            pltpu.SemaphoreType.DMA,
        ],
    )(_sc_body)
    return f(pred_sc, lab_sc, score)


def _tc_body(score_ref, pred_ref, lab_ref, out_ref):
    sc = score_ref[...]

    acc = jnp.zeros((SUB, C), jnp.float32)
    for j in range(BR // SUB):
        x = pred_ref[pl.ds(j * SUB, SUB), :]
        pos = lab_ref[pl.ds(j * SUB, SUB), :] > 0
        ax = jnp.abs(x)
        u = jnp.exp(-ax)
        d = 1.0 / (1.0 + u)
        sp = jnp.maximum(x, 0.0) + jnp.log1p(u)
        s = jnp.where(x >= 0.0, d, 1.0 - d)
        scm = jnp.where(pos, sc, 0.0)
        a = jnp.where(pos, jnp.abs(sc - s), s)
        acc = acc + (sp - x * scm) * (a * lax.sqrt(a))

    @pl.when(pl.program_id(0) == 0)
    def _():
        out_ref[0, 0] = 0.0

    out_ref[0, 0] += jnp.sum(acc)


def _tc_call(pred, label, score):
    return pl.pallas_call(
        _tc_body,
        grid=(G_TC,),
        in_specs=[
            pl.BlockSpec((1, C), lambda i: (0, 0)),
            pl.BlockSpec((BR, C), lambda i: (i, 0)),
            pl.BlockSpec((BR, C), lambda i: (i, 0)),
        ],
        out_specs=pl.BlockSpec((1, 1), lambda i: (0, 0),
                               memory_space=pltpu.SMEM),
        out_shape=jax.ShapeDtypeStruct((1, 1), jnp.float32),
    )(score.reshape(1, C), pred, label)


@jax.jit
def kernel(pred, label, score):
    sc_part = _sc_call(pred[R_TC:], label[R_TC:], score)
    tc_part = _tc_call(pred, label, score)
    return (jnp.sum(tc_part) + jnp.sum(sc_part)) / jnp.float32(TOTAL)


# speedup vs baseline: 1.2120x; 1.0269x over previous
import functools

import jax
import jax.numpy as jnp
from jax import lax
from jax.experimental import pallas as pl
from jax.experimental.pallas import tpu as pltpu
from jax.experimental.pallas import tpu_sc as plsc

N = 65536
C = 80
TOTAL = N * C
L = 16
VPR = C // L

R_SC = 8192
R_TC = N - R_SC
NW = 32
RPW = R_SC // NW
CROWS = 128
NCH = RPW // CROWS

BR = 4096
G_TC = R_TC // BR
SUB = 32

_LOG1P_C = (1.6936626598407223e-06, 0.9998325947816316, -0.49720333122019134,
            0.31504127990864345, -0.18901954822291905, 0.08152317761736225,
            -0.017029610589052675)


def _log1p01(u):
    p = jnp.float32(_LOG1P_C[6])
    for c in _LOG1P_C[5::-1]:
        p = p * u + jnp.float32(c)
    return p


def _pow15(a):
    i = lax.bitcast_convert_type(a, jnp.int32)
    y = lax.bitcast_convert_type(
        jnp.int32(0x5F3759DF) - lax.shift_right_arithmetic(i, 1), jnp.float32)
    y = y * (1.5 - 0.5 * a * y * y)
    y = y * (1.5 - 0.5 * a * y * y)
    return a * a * y


def _elem(x, pos, sc):
    ax = jnp.abs(x)
    u = jnp.exp(-ax)
    d = 1.0 / (1.0 + u)
    sp = jnp.maximum(x, 0.0) + _log1p01(u)
    s = jnp.where(x >= 0.0, d, 1.0 - d)
    scm = jnp.where(pos, sc, 0.0)
    a = jnp.where(pos, jnp.abs(sc - s), s)
    return (sp - x * scm) * _pow15(a)


def _sc_body(pred_h, lab_h, score_h, out_h,
             pb0, pb1, lb0, lb1, sc_v, acc_v,
             sp0, sp1, sl0, sl1):
    wid = lax.axis_index("s") * 2 + lax.axis_index("c")
    base = wid * RPW
    last = base + (NCH - 1) * CROWS

    pltpu.sync_copy(score_h, sc_v)
    scv = [sc_v[pl.ds(L * v, L)] for v in range(VPR)]

    def start(row0, pb, lb, sp, sl):
        row = jnp.minimum(row0, last)
        pltpu.async_copy(pred_h.at[pl.ds(row, CROWS)], pb, sp)
        pltpu.async_copy(lab_h.at[pl.ds(row, CROWS)], lb, sl)

    def wait(pb, lb, sp, sl):
        pltpu.make_async_copy(pred_h.at[pl.ds(base, CROWS)], pb, sp).wait()
        pltpu.make_async_copy(lab_h.at[pl.ds(base, CROWS)], lb, sl).wait()

    def compute(pb, lb, acc):
        def row(r, acc):
            for v in range(VPR):
                x = pb[r, pl.ds(L * v, L)]
                lv = lb[r, pl.ds(L * v, L)]
                acc = acc + _elem(x, lv > 0, scv[v])
            return acc
        return lax.fori_loop(0, CROWS, row, acc)

    start(base, pb0, lb0, sp0, sl0)
    start(base + CROWS, pb1, lb1, sp1, sl1)

    def pair(g, acc):
        c0 = base + (2 * g) * CROWS
        wait(pb0, lb0, sp0, sl0)
        acc = compute(pb0, lb0, acc)
        start(c0 + 2 * CROWS, pb0, lb0, sp0, sl0)
        wait(pb1, lb1, sp1, sl1)
        acc = compute(pb1, lb1, acc)
        start(c0 + 3 * CROWS, pb1, lb1, sp1, sl1)
        return acc

    acc = lax.fori_loop(0, NCH // 2, pair, jnp.zeros((L,), jnp.float32))

    if NCH % 2:
        wait(pb0, lb0, sp0, sl0)
        acc = compute(pb0, lb0, acc)
        start(last, pb0, lb0, sp0, sl0)

    wait(pb0, lb0, sp0, sl0)
    wait(pb1, lb1, sp1, sl1)

    acc_v[...] = acc
    pltpu.sync_copy(acc_v, out_h.at[pl.ds(wid * L, L)])


def _sc_call(pred_sc, lab_sc, score):
    mesh = plsc.VectorSubcoreMesh(core_axis_name="c", subcore_axis_name="s")
    f = functools.partial(
        pl.kernel,
        mesh=mesh,
        out_type=jax.ShapeDtypeStruct((NW * L,), jnp.float32),
        compiler_params=pltpu.CompilerParams(use_tc_tiling_on_sc=True),
        scratch_types=[
            pltpu.VMEM((CROWS, C), jnp.float32),
            pltpu.VMEM((CROWS, C), jnp.float32),
            pltpu.VMEM((CROWS, C), jnp.int32),
            pltpu.VMEM((CROWS, C), jnp.int32),
            pltpu.VMEM((C,), jnp.float32),
            pltpu.VMEM((L,), jnp.float32),
            pltpu.SemaphoreType.DMA,
            pltpu.SemaphoreType.DMA,
            pltpu.SemaphoreType.DMA,
            pltpu.SemaphoreType.DMA,
        ],
    )(_sc_body)
    return f(pred_sc, lab_sc, score)


def _tc_body(score_ref, pred_ref, lab_ref, out_ref):
    sc = score_ref[...]

    acc = jnp.zeros((SUB, C), jnp.float32)
    for j in range(BR // SUB):
        x = pred_ref[pl.ds(j * SUB, SUB), :]
        pos = lab_ref[pl.ds(j * SUB, SUB), :] > 0
        ax = jnp.abs(x)
        u = jnp.exp(-ax)
        d = 1.0 / (1.0 + u)
        sp = jnp.maximum(x, 0.0) + jnp.log1p(u)
        s = jnp.where(x >= 0.0, d, 1.0 - d)
        scm = jnp.where(pos, sc, 0.0)
        a = jnp.where(pos, jnp.abs(sc - s), s)
        acc = acc + (sp - x * scm) * (a * lax.sqrt(a))

    @pl.when(pl.program_id(0) == 0)
    def _():
        out_ref[0, 0] = 0.0

    out_ref[0, 0] += jnp.sum(acc)


def _tc_call(pred, label, score):
    return pl.pallas_call(
        _tc_body,
        grid=(G_TC,),
        in_specs=[
            pl.BlockSpec((1, C), lambda i: (0, 0)),
            pl.BlockSpec((BR, C), lambda i: (i, 0)),
            pl.BlockSpec((BR, C), lambda i: (i, 0)),
        ],
        out_specs=pl.BlockSpec((1, 1), lambda i: (0, 0),
                               memory_space=pltpu.SMEM),
        out_shape=jax.ShapeDtypeStruct((1, 1), jnp.float32),
    )(score.reshape(1, C), pred, label)


@jax.jit
def kernel(pred, label, score):
    sc_part = _sc_call(pred[R_TC:], label[R_TC:], score)
    tc_part = _tc_call(pred, label, score)
    return (jnp.sum(tc_part) + jnp.sum(sc_part)) / jnp.float32(TOTAL)


# speedup vs baseline: 1.2605x; 1.0400x over previous
import functools

import jax
import jax.numpy as jnp
from jax import lax
from jax.experimental import pallas as pl
from jax.experimental.pallas import tpu as pltpu
from jax.experimental.pallas import tpu_sc as plsc

N = 65536
C = 80
TOTAL = N * C
L = 16
VPR = C // L
W = 128

R_SC = 12288
R_TC = N - R_SC
NW = 32
RPW = R_SC // NW
CROWS = 128
NCH = RPW // CROWS

BR = 4096
G_TC = R_TC // BR
G_ST = R_SC // BR
SUB = 32

_LOG1P_C = (1.6936626598407223e-06, 0.9998325947816316, -0.49720333122019134,
            0.31504127990864345, -0.18901954822291905, 0.08152317761736225,
            -0.017029610589052675)


def _log1p01(u):
    p = jnp.float32(_LOG1P_C[6])
    for c in _LOG1P_C[5::-1]:
        p = p * u + jnp.float32(c)
    return p


def _pow15(a):
    i = lax.bitcast_convert_type(a, jnp.int32)
    y = lax.bitcast_convert_type(
        jnp.int32(0x5F3759DF) - lax.shift_right_arithmetic(i, 1), jnp.float32)
    y = y * (1.5 - 0.5 * a * y * y)
    y = y * (1.5 - 0.5 * a * y * y)
    return a * a * y


def _elem(x, pos, sc):
    ax = jnp.abs(x)
    u = jnp.exp(-ax)
    d = 1.0 / (1.0 + u)
    sp = jnp.maximum(x, 0.0) + _log1p01(u)
    s = jnp.where(x >= 0.0, d, 1.0 - d)
    scm = jnp.where(pos, sc, 0.0)
    a = jnp.where(pos, jnp.abs(sc - s), s)
    return (sp - x * scm) * _pow15(a)


def _stage_body(pred_ref, lab_ref, out_ref):
    x = pred_ref[...]
    xi = lax.bitcast_convert_type(x, jnp.int32)
    posb = (lab_ref[...] > 0).astype(jnp.int32)
    out_ref[:, :C] = lax.bitcast_convert_type((xi & -2) | posb, jnp.float32)


def _stage_call(pred, label):
    return pl.pallas_call(
        _stage_body,
        grid=(G_ST,),
        in_specs=[
            pl.BlockSpec((BR, C), lambda i: (G_TC + i, 0)),
            pl.BlockSpec((BR, C), lambda i: (G_TC + i, 0)),
        ],
        out_specs=pl.BlockSpec((BR, W), lambda i: (i, 0)),
        out_shape=jax.ShapeDtypeStruct((R_SC, W), jnp.float32),
    )(pred, label)


def _sc_body(xe_h, score_h, out_h, pb0, pb1, sc_v, acc_v, sp0, sp1):
    wid = lax.axis_index("s") * 2 + lax.axis_index("c")
    base = wid * RPW
    last = base + (NCH - 1) * CROWS

    pltpu.sync_copy(score_h, sc_v)
    scv = [sc_v[pl.ds(L * v, L)] for v in range(VPR)]

    def start(row0, pb, sp):
        row = jnp.minimum(row0, last)
        pltpu.async_copy(xe_h.at[pl.ds(row, CROWS)], pb, sp)

    def wait(pb, sp):
        pltpu.make_async_copy(xe_h.at[pl.ds(base, CROWS)], pb, sp).wait()

    def compute(pb, acc):
        def row(r, acc):
            for v in range(VPR):
                xe = pb[r, pl.ds(L * v, L)]
                pos = (lax.bitcast_convert_type(xe, jnp.int32) & 1) > 0
                acc = acc + _elem(xe, pos, scv[v])
            return acc
        return lax.fori_loop(0, CROWS, row, acc)

    start(base, pb0, sp0)
    start(base + CROWS, pb1, sp1)

    def pair(g, acc):
        c0 = base + (2 * g) * CROWS
        wait(pb0, sp0)
        acc = compute(pb0, acc)
        start(c0 + 2 * CROWS, pb0, sp0)
        wait(pb1, sp1)
        acc = compute(pb1, acc)
        start(c0 + 3 * CROWS, pb1, sp1)
        return acc

    acc = lax.fori_loop(0, NCH // 2, pair, jnp.zeros((L,), jnp.float32))

    if NCH % 2:
        wait(pb0, sp0)
        acc = compute(pb0, acc)
        start(last, pb0, sp0)

    wait(pb0, sp0)
    wait(pb1, sp1)

    acc_v[...] = acc
    pltpu.sync_copy(acc_v, out_h.at[pl.ds(wid * L, L)])


def _sc_call(xe, score):
    mesh = plsc.VectorSubcoreMesh(core_axis_name="c", subcore_axis_name="s")
    f = functools.partial(
        pl.kernel,
        mesh=mesh,
        out_type=jax.ShapeDtypeStruct((NW * L,), jnp.float32),
        scratch_types=[
            pltpu.VMEM((CROWS, W), jnp.float32),
            pltpu.VMEM((CROWS, W), jnp.float32),
            pltpu.VMEM((C,), jnp.float32),
            pltpu.VMEM((L,), jnp.float32),
            pltpu.SemaphoreType.DMA,
            pltpu.SemaphoreType.DMA,
        ],
    )(_sc_body)
    return f(xe, score)


def _tc_body(score_ref, pred_ref, lab_ref, out_ref):
    sc = score_ref[...]

    acc = jnp.zeros((SUB, C), jnp.float32)
    for j in range(BR // SUB):
        x = pred_ref[pl.ds(j * SUB, SUB), :]
        pos = lab_ref[pl.ds(j * SUB, SUB), :] > 0
        ax = jnp.abs(x)
        u = jnp.exp(-ax)
        d = 1.0 / (1.0 + u)
        sp = jnp.maximum(x, 0.0) + jnp.log1p(u)
        s = jnp.where(x >= 0.0, d, 1.0 - d)
        scm = jnp.where(pos, sc, 0.0)
        a = jnp.where(pos, jnp.abs(sc - s), s)
        acc = acc + (sp - x * scm) * (a * lax.sqrt(a))

    @pl.when(pl.program_id(0) == 0)
    def _():
        out_ref[0, 0] = 0.0

    out_ref[0, 0] += jnp.sum(acc)


def _tc_call(pred, label, score):
    return pl.pallas_call(
        _tc_body,
        grid=(G_TC,),
        in_specs=[
            pl.BlockSpec((1, C), lambda i: (0, 0)),
            pl.BlockSpec((BR, C), lambda i: (i, 0)),
            pl.BlockSpec((BR, C), lambda i: (i, 0)),
        ],
        out_specs=pl.BlockSpec((1, 1), lambda i: (0, 0),
                               memory_space=pltpu.SMEM),
        out_shape=jax.ShapeDtypeStruct((1, 1), jnp.float32),
    )(score.reshape(1, C), pred, label)


@jax.jit
def kernel(pred, label, score):
    xe = _stage_call(pred, label)
    sc_part = _sc_call(xe, score)
    tc_part = _tc_call(pred, label, score)
    return (jnp.sum(tc_part) + jnp.sum(sc_part)) / jnp.float32(TOTAL)


# speedup vs baseline: 2.6257x; 2.0831x over previous
import functools

import jax
import jax.numpy as jnp
from jax import lax
from jax.experimental import pallas as pl
from jax.experimental.pallas import tpu as pltpu
from jax.experimental.pallas import tpu_sc as plsc

N = 65536
C = 80
TOTAL = N * C
L = 16

R_SC = 8192
R_TC = N - R_SC
NW = 32
CK = R_SC // NW
NV = CK // L

BLK = 4096
G_TC = R_TC // BLK
SUBC = 512

_LOG1P_C = (1.6936626598407223e-06, 0.9998325947816316, -0.49720333122019134,
            0.31504127990864345, -0.18901954822291905, 0.08152317761736225,
            -0.017029610589052675)


def _log1p01(u):
    p = jnp.float32(_LOG1P_C[6])
    for c in _LOG1P_C[5::-1]:
        p = p * u + jnp.float32(c)
    return p


def _pow15_sc(a):
    i = lax.bitcast_convert_type(a, jnp.int32)
    y = lax.bitcast_convert_type(
        jnp.int32(0x5F3759DF) - lax.shift_right_arithmetic(i, 1), jnp.float32)
    y = y * (1.5 - 0.5 * a * y * y)
    y = y * (1.5 - 0.5 * a * y * y)
    return a * a * y


def _loss_tc(x, pos, sc):
    ax = jnp.abs(x)
    u = jnp.exp(-ax)
    d = 1.0 / (1.0 + u)
    sp = jnp.maximum(x, 0.0) + jnp.log1p(u)
    s = jnp.where(x >= 0.0, d, 1.0 - d)
    scm = jnp.where(pos, sc, 0.0)
    a = jnp.where(pos, jnp.abs(sc - s), s)
    return (sp - x * scm) * (a * lax.sqrt(a))


def _loss_sc(x, pos, sc):
    ax = jnp.abs(x)
    u = jnp.exp(-ax)
    d = 1.0 / (1.0 + u)
    sp = jnp.maximum(x, 0.0) + _log1p01(u)
    s = jnp.where(x >= 0.0, d, 1.0 - d)
    scm = jnp.where(pos, sc, 0.0)
    a = jnp.where(pos, jnp.abs(sc - s), s)
    return (sp - x * scm) * _pow15_sc(a)


def _sc_body(pred_h, lab_h, scb_h, out_h, pb, lb, sc_v, acc_v, sem):
    wid = lax.axis_index("s") * 2 + lax.axis_index("c")
    col0 = wid * CK

    pltpu.sync_copy(scb_h, sc_v)
    cp_p = pltpu.async_copy(pred_h.at[:, pl.ds(col0, CK)], pb, sem)
    cp_p.wait()
    cp_l = pltpu.async_copy(lab_h.at[:, pl.ds(col0, CK)], lb, sem)
    cp_l.wait()

    def cls(c, acc):
        sc = sc_v[c, :]

        def vec(k, acc):
            x = pb[c, pl.ds(k * L, L)]
            lv = lb[c, pl.ds(k * L, L)]
            return acc + _loss_sc(x, lv > 0, sc)

        return lax.fori_loop(0, NV, vec, acc)

    acc = lax.fori_loop(0, C, cls, jnp.zeros((L,), jnp.float32))

    acc_v[...] = acc
    pltpu.sync_copy(acc_v, out_h.at[pl.ds(wid * L, L)])


def _sc_call(pred_sc, lab_sc, score_b):
    mesh = plsc.VectorSubcoreMesh(core_axis_name="c", subcore_axis_name="s")
    f = functools.partial(
        pl.kernel,
        mesh=mesh,
        out_type=jax.ShapeDtypeStruct((NW * L,), jnp.float32),
        scratch_types=[
            pltpu.VMEM((C, CK), jnp.float32),
            pltpu.VMEM((C, CK), jnp.int32),
            pltpu.VMEM((C, L), jnp.float32),
            pltpu.VMEM((L,), jnp.float32),
            pltpu.SemaphoreType.DMA,
        ],
    )(_sc_body)
    return f(pred_sc, lab_sc, score_b)


def _tc_body(score_ref, pred_ref, lab_ref, out_ref):
    acc = jnp.zeros((8, SUBC), jnp.float32)
    for j in range(C // 8):
        sc = score_ref[pl.ds(8 * j, 8), :]
        for k in range(BLK // SUBC):
            x = pred_ref[pl.ds(8 * j, 8), pl.ds(k * SUBC, SUBC)]
            pos = lab_ref[pl.ds(8 * j, 8), pl.ds(k * SUBC, SUBC)] > 0
            acc = acc + _loss_tc(x, pos, sc)

    @pl.when(pl.program_id(0) == 0)
    def _():
        out_ref[0, 0] = 0.0

    out_ref[0, 0] += jnp.sum(acc)


def _tc_call(predT, labT, score):
    return pl.pallas_call(
        _tc_body,
        grid=(G_TC,),
        in_specs=[
            pl.BlockSpec((C, 1), lambda i: (0, 0)),
            pl.BlockSpec((C, BLK), lambda i: (0, i)),
            pl.BlockSpec((C, BLK), lambda i: (0, i)),
        ],
        out_specs=pl.BlockSpec((1, 1), lambda i: (0, 0),
                               memory_space=pltpu.SMEM),
        out_shape=jax.ShapeDtypeStruct((1, 1), jnp.float32),
    )(score.reshape(C, 1), predT, labT)


@jax.jit
def kernel(pred, label, score):
    predT = pred.T
    labT = label.T
    score_b = jnp.tile(score.reshape(C, 1), (1, L))
    sc_part = _sc_call(predT[:, R_TC:], labT[:, R_TC:], score_b)
    tc_part = _tc_call(predT, labT, score)
    return (jnp.sum(tc_part) + jnp.sum(sc_part)) / jnp.float32(TOTAL)


# speedup vs baseline: 2.6371x; 1.0043x over previous
import functools

import jax
import jax.numpy as jnp
from jax import lax
from jax.experimental import pallas as pl
from jax.experimental.pallas import tpu as pltpu
from jax.experimental.pallas import tpu_sc as plsc

N = 65536
C = 80
TOTAL = N * C
L = 16

R_SC = 8192
R_TC = N - R_SC
NW = 32
CK = R_SC // NW
NV = CK // L

BLK = 8192
G_TC = R_TC // BLK
SUBC = 512

_LOG1P_C = (1.6936626598407223e-06, 0.9998325947816316, -0.49720333122019134,
            0.31504127990864345, -0.18901954822291905, 0.08152317761736225,
            -0.017029610589052675)


def _log1p01(u):
    p = jnp.float32(_LOG1P_C[6])
    for c in _LOG1P_C[5::-1]:
        p = p * u + jnp.float32(c)
    return p


def _pow15_sc(a):
    i = lax.bitcast_convert_type(a, jnp.int32)
    y = lax.bitcast_convert_type(
        jnp.int32(0x5F3759DF) - lax.shift_right_arithmetic(i, 1), jnp.float32)
    y = y * (1.5 - 0.5 * a * y * y)
    y = y * (1.5 - 0.5 * a * y * y)
    return a * a * y


def _loss_tc(x, pos, sc):
    ax = jnp.abs(x)
    u = jnp.exp(-ax)
    d = 1.0 / (1.0 + u)
    sp = jnp.maximum(x, 0.0) + jnp.log1p(u)
    s = jnp.where(x >= 0.0, d, 1.0 - d)
    scm = jnp.where(pos, sc, 0.0)
    a = jnp.where(pos, jnp.abs(sc - s), s)
    return (sp - x * scm) * (a * lax.sqrt(a))


def _loss_sc(x, pos, sc):
    ax = jnp.abs(x)
    u = jnp.exp(-ax)
    d = 1.0 / (1.0 + u)
    sp = jnp.maximum(x, 0.0) + _log1p01(u)
    s = jnp.where(x >= 0.0, d, 1.0 - d)
    scm = jnp.where(pos, sc, 0.0)
    a = jnp.where(pos, jnp.abs(sc - s), s)
    return (sp - x * scm) * _pow15_sc(a)


def _sc_body(pred_h, lab_h, scb_h, out_h, pb, lb, sc_v, acc_v, sem, sem2):
    wid = lax.axis_index("s") * 2 + lax.axis_index("c")
    col0 = wid * CK

    pltpu.sync_copy(scb_h, sc_v)
    cp_p = pltpu.async_copy(pred_h.at[:, pl.ds(col0, CK)], pb, sem)
    cp_l = pltpu.async_copy(lab_h.at[:, pl.ds(col0, CK)], lb, sem2)
    cp_p.wait()
    cp_l.wait()

    def cls(c, acc):
        sc = sc_v[c, :]

        def vec(k, acc):
            for kk in range(4):
                x = pb[c, pl.ds((k * 4 + kk) * L, L)]
                lv = lb[c, pl.ds((k * 4 + kk) * L, L)]
                acc = acc + _loss_sc(x, lv > 0, sc)
            return acc

        return lax.fori_loop(0, NV // 4, vec, acc)

    acc = lax.fori_loop(0, C, cls, jnp.zeros((L,), jnp.float32))

    acc_v[...] = acc
    pltpu.sync_copy(acc_v, out_h.at[pl.ds(wid * L, L)])


def _sc_call(pred_sc, lab_sc, score_b):
    mesh = plsc.VectorSubcoreMesh(core_axis_name="c", subcore_axis_name="s")
    f = functools.partial(
        pl.kernel,
        mesh=mesh,
        out_type=jax.ShapeDtypeStruct((NW * L,), jnp.float32),
        scratch_types=[
            pltpu.VMEM((C, CK), jnp.float32),
            pltpu.VMEM((C, CK), jnp.int32),
            pltpu.VMEM((C, L), jnp.float32),
            pltpu.VMEM((L,), jnp.float32),
            pltpu.SemaphoreType.DMA,
            pltpu.SemaphoreType.DMA,
        ],
    )(_sc_body)
    return f(pred_sc, lab_sc, score_b)


def _tc_body(score_ref, pred_ref, lab_ref, out_ref):
    acc = jnp.zeros((8, SUBC), jnp.float32)
    for j in range(C // 8):
        sc = score_ref[pl.ds(8 * j, 8), :]
        for k in range(BLK // SUBC):
            x = pred_ref[pl.ds(8 * j, 8), pl.ds(k * SUBC, SUBC)]
            pos = lab_ref[pl.ds(8 * j, 8), pl.ds(k * SUBC, SUBC)] > 0
            acc = acc + _loss_tc(x, pos, sc)

    @pl.when(pl.program_id(0) == 0)
    def _():
        out_ref[0, 0] = 0.0

    out_ref[0, 0] += jnp.sum(acc)


def _tc_call(predT, labT, score):
    return pl.pallas_call(
        _tc_body,
        grid=(G_TC,),
        in_specs=[
            pl.BlockSpec((C, 1), lambda i: (0, 0)),
            pl.BlockSpec((C, BLK), lambda i: (0, i)),
            pl.BlockSpec((C, BLK), lambda i: (0, i)),
        ],
        out_specs=pl.BlockSpec((1, 1), lambda i: (0, 0),
                               memory_space=pltpu.SMEM),
        out_shape=jax.ShapeDtypeStruct((1, 1), jnp.float32),
    )(score.reshape(C, 1), predT, labT)


@jax.jit
def kernel(pred, label, score):
    predT = pred.T
    labT = label.T
    score_b = jnp.tile(score.reshape(C, 1), (1, L))
    sc_part = _sc_call(predT[:, R_TC:], labT[:, R_TC:], score_b)
    tc_part = _tc_call(predT, labT, score)
    return (jnp.sum(tc_part) + jnp.sum(sc_part)) / jnp.float32(TOTAL)
